# baseline (speedup 1.0000x reference)
import functools

import jax
import jax.numpy as jnp
from jax import lax
from jax.experimental import pallas as pl
from jax.experimental.pallas import tpu as pltpu
from jax.experimental.pallas import tpu_sc as plsc

N = 10000
E = 320000
D = 128
H = 128
L = 3
G = 128

NC = 2
NS = 16
NW = NC * NS

K = 128
NCHUNK0 = 108
NCHUNK1 = 50
NCHUNK_MAX = NCHUNK0
E_CAP = NS * (NCHUNK0 + NCHUNK1) * K
N_ACC = 10112
ACC_ROWS_PER_TILE = N_ACC // NS


def _unpack_chunk(packed_v, a, idx_v, r):
    for kk in range(K // 16):
        w = packed_v[a, pl.ds(kk * 16, 16)]
        idx_v[r, pl.ds(kk * 16, 16)] = w & 0xFFFF
        idx_v[r + 1, pl.ds(kk * 16, 16)] = lax.shift_right_logical(w, 16)


def _sc_scatter_body(h_hbm, packed_hbm, zeros_hbm, out_hbm,
                     acc, packed_v, idx_v, rows0_v, rows1_v, sem0, sem1):
    c = lax.axis_index("c")
    s = lax.axis_index("s")
    wid = c * NS + s

    pltpu.sync_copy(zeros_hbm.at[pl.ds(s * ACC_ROWS_PER_TILE, ACC_ROWS_PER_TILE)],
                    acc.at[pl.ds(s * ACC_ROWS_PER_TILE, ACC_ROWS_PER_TILE)])
    plsc.subcore_barrier()

    pltpu.sync_copy(packed_hbm.at[wid], packed_v)

    npair = jnp.where(c == 0, NCHUNK0 // 2, NCHUNK1 // 2)
    _unpack_chunk(packed_v, 0, idx_v, 0)
    pltpu.async_copy(h_hbm.at[idx_v.at[0]], rows0_v, sem0)

    def pair_body(i, carry):
        a = 2 * i
        _unpack_chunk(packed_v, a + 1, idx_v, 2)
        pltpu.make_async_copy(h_hbm.at[idx_v.at[0]], rows0_v, sem0).wait()
        pltpu.async_copy(h_hbm.at[idx_v.at[2]], rows1_v, sem1)
        pltpu.sync_copy(rows0_v, acc.at[idx_v.at[1]], add=True)
        _unpack_chunk(packed_v, a + 2, idx_v, 0)
        pltpu.make_async_copy(h_hbm.at[idx_v.at[2]], rows1_v, sem1).wait()
        pltpu.async_copy(h_hbm.at[idx_v.at[0]], rows0_v, sem0)
        pltpu.sync_copy(rows1_v, acc.at[idx_v.at[3]], add=True)
        return carry

    lax.fori_loop(0, npair - 1, pair_body, 0, unroll=False)

    a = 2 * (npair - 1)
    _unpack_chunk(packed_v, a + 1, idx_v, 2)
    pltpu.make_async_copy(h_hbm.at[idx_v.at[0]], rows0_v, sem0).wait()
    pltpu.async_copy(h_hbm.at[idx_v.at[2]], rows1_v, sem1)
    pltpu.sync_copy(rows0_v, acc.at[idx_v.at[1]], add=True)
    pltpu.make_async_copy(h_hbm.at[idx_v.at[2]], rows1_v, sem1).wait()
    pltpu.sync_copy(rows1_v, acc.at[idx_v.at[3]], add=True)
    plsc.subcore_barrier()

    pltpu.sync_copy(acc.at[pl.ds(s * ACC_ROWS_PER_TILE, ACC_ROWS_PER_TILE)],
                    out_hbm.at[c, pl.ds(s * ACC_ROWS_PER_TILE, ACC_ROWS_PER_TILE)])


_sc_scatter = pl.kernel(
    _sc_scatter_body,
    out_type=jax.ShapeDtypeStruct((NC, N_ACC, H), jnp.float32),
    mesh=plsc.VectorSubcoreMesh(core_axis_name="c", subcore_axis_name="s"),
    scratch_types=[
        pltpu.VMEM_SHARED((N_ACC, H), jnp.float32),
        pltpu.VMEM((NCHUNK_MAX, K), jnp.int32),
        pltpu.VMEM((8, K), jnp.int32),
        pltpu.VMEM((K, H), jnp.float32),
        pltpu.VMEM((K, H), jnp.float32),
        pltpu.SemaphoreType.DMA,
        pltpu.SemaphoreType.DMA,
    ],
)


def _tc_dense_body(p_ref, h_ref, w1_ref, b1_ref, w2_ref, b2_ref,
                   gam_ref, bet_ref, batch_ref, t_ref, pool_ref):
    m = p_ref[0, :N] + p_ref[1, :N] + h_ref[...]
    t1 = jnp.dot(m, w1_ref[...], preferred_element_type=jnp.float32) + b1_ref[...]
    t1 = jnp.maximum(t1, 0.0)
    t2 = jnp.dot(t1, w2_ref[...], preferred_element_type=jnp.float32) + b2_ref[...]
    t2 = jnp.maximum(t2, 0.0)
    mu = jnp.mean(t2, axis=0, keepdims=True)
    var = jnp.mean((t2 - mu) * (t2 - mu), axis=0, keepdims=True)
    tn = (t2 - mu) / jnp.sqrt(var + 1e-5) * gam_ref[...] + bet_ref[...]
    t_ref[...] = tn
    onehot = (batch_ref[...] == lax.broadcasted_iota(jnp.int32, (N, G), 1)
              ).astype(jnp.float32)
    pool_ref[...] = lax.dot_general(onehot, tn, (((0,), (0,)), ((), ())),
                                    preferred_element_type=jnp.float32)


_tc_dense = pl.pallas_call(
    _tc_dense_body,
    out_shape=(
        jax.ShapeDtypeStruct((N, H), jnp.float32),
        jax.ShapeDtypeStruct((G, H), jnp.float32),
    ),
)


def kernel(x, edge_index, batch, num_graphs, W1, b1, W2, b2, gamma, beta):
    src = edge_index[0]
    dst = edge_index[1]
    packed = jnp.bitwise_or(src, jnp.left_shift(dst, 16))
    fill = jnp.int32(N << 16)
    p = jnp.concatenate([packed, jnp.full((E_CAP - E,), fill, jnp.int32)])
    n0 = NS * NCHUNK0 * K
    c0 = p[:n0].reshape(NS, NCHUNK0, K)
    c1 = p[n0:].reshape(NS, NCHUNK1, K)
    c1 = jnp.pad(c1, ((0, 0), (0, NCHUNK_MAX - NCHUNK1), (0, 0)),
                 constant_values=fill)
    packed_p = jnp.concatenate([c0, c1], axis=0)
    zeros = jnp.zeros((N_ACC, H), jnp.float32)
    batch2d = batch.reshape(N, 1)

    h = x
    pools = []
    for i in range(L):
        partials = _sc_scatter(h, packed_p, zeros)
        t, pool = _tc_dense(partials, h, W1[i], b1[i].reshape(1, H),
                            W2[i], b2[i].reshape(1, H),
                            gamma[i].reshape(1, H), beta[i].reshape(1, H),
                            batch2d)
        pools.append(pool)
        h = t
    return jnp.concatenate(pools, axis=1)


# speedup vs baseline: 1.0749x; 1.0749x over previous
import functools

import jax
import jax.numpy as jnp
from jax import lax
from jax.experimental import pallas as pl
from jax.experimental.pallas import tpu as pltpu
from jax.experimental.pallas import tpu_sc as plsc

N = 10000
E = 320000
D = 128
H = 128
L = 3
G = 128

NC = 2
NS = 16
NW = NC * NS

K = 128
NCHUNK0 = 104
NCHUNK1 = 53
NCHUNK_MAX = NCHUNK0
E_CAP = NS * (NCHUNK0 + NCHUNK1) * K
N_ACC = 10112
ACC_ROWS_PER_TILE = N_ACC // NS


def _sc_scatter_body(h_hbm, src_hbm, dst_hbm, zeros_hbm, out_hbm,
                     acc, src_v, dst_v, rows0_v, sem0):
    c = lax.axis_index("c")
    s = lax.axis_index("s")
    wid = c * NS + s

    pltpu.sync_copy(zeros_hbm.at[pl.ds(s * ACC_ROWS_PER_TILE, ACC_ROWS_PER_TILE)],
                    acc.at[pl.ds(s * ACC_ROWS_PER_TILE, ACC_ROWS_PER_TILE)])
    plsc.subcore_barrier()

    pltpu.sync_copy(src_hbm.at[wid], src_v)
    pltpu.sync_copy(dst_hbm.at[wid], dst_v)

    def chunk_body(j, carry):
        pltpu.async_copy(h_hbm.at[src_v.at[j]], rows0_v, sem0).wait()
        pltpu.sync_copy(rows0_v, acc.at[dst_v.at[j]], add=True)
        return carry

    nch = jnp.where(c == 0, NCHUNK0, NCHUNK1)
    lax.fori_loop(0, nch, chunk_body, 0, unroll=False)
    plsc.subcore_barrier()

    pltpu.sync_copy(acc.at[pl.ds(s * ACC_ROWS_PER_TILE, ACC_ROWS_PER_TILE)],
                    out_hbm.at[c, pl.ds(s * ACC_ROWS_PER_TILE, ACC_ROWS_PER_TILE)])


_sc_scatter = pl.kernel(
    _sc_scatter_body,
    out_type=jax.ShapeDtypeStruct((NC, N_ACC, H), jnp.float32),
    mesh=plsc.VectorSubcoreMesh(core_axis_name="c", subcore_axis_name="s"),
    scratch_types=[
        pltpu.VMEM_SHARED((N_ACC, H), jnp.float32),
        pltpu.VMEM((NCHUNK_MAX, K), jnp.int32),
        pltpu.VMEM((NCHUNK_MAX, K), jnp.int32),
        pltpu.VMEM((K, H), jnp.float32),
        pltpu.SemaphoreType.DMA,
    ],
)


def _tc_dense_body(p_ref, h_ref, w1_ref, b1_ref, w2_ref, b2_ref,
                   gam_ref, bet_ref, batch_ref, t_ref, pool_ref):
    m = p_ref[0, :N] + p_ref[1, :N] + h_ref[...]
    t1 = jnp.dot(m, w1_ref[...], preferred_element_type=jnp.float32) + b1_ref[...]
    t1 = jnp.maximum(t1, 0.0)
    t2 = jnp.dot(t1, w2_ref[...], preferred_element_type=jnp.float32) + b2_ref[...]
    t2 = jnp.maximum(t2, 0.0)
    mu = jnp.mean(t2, axis=0, keepdims=True)
    var = jnp.mean((t2 - mu) * (t2 - mu), axis=0, keepdims=True)
    tn = (t2 - mu) / jnp.sqrt(var + 1e-5) * gam_ref[...] + bet_ref[...]
    t_ref[...] = tn
    onehot = (batch_ref[...] == lax.broadcasted_iota(jnp.int32, (N, G), 1)
              ).astype(jnp.float32)
    pool_ref[...] = lax.dot_general(onehot, tn, (((0,), (0,)), ((), ())),
                                    preferred_element_type=jnp.float32)


_tc_dense = pl.pallas_call(
    _tc_dense_body,
    out_shape=(
        jax.ShapeDtypeStruct((N, H), jnp.float32),
        jax.ShapeDtypeStruct((G, H), jnp.float32),
    ),
)


def kernel(x, edge_index, batch, num_graphs, W1, b1, W2, b2, gamma, beta):
    src = edge_index[0]
    dst = edge_index[1]

    def layout(idx, fill):
        p = jnp.concatenate([idx, jnp.full((E_CAP - E,), fill, jnp.int32)])
        n0 = NS * NCHUNK0 * K
        c0 = p[:n0].reshape(NS, NCHUNK0, K)
        c1 = p[n0:].reshape(NS, NCHUNK1, K)
        c1 = jnp.pad(c1, ((0, 0), (0, NCHUNK_MAX - NCHUNK1), (0, 0)),
                     constant_values=fill)
        return jnp.concatenate([c0, c1], axis=0)

    src_p = layout(src, 0)
    dst_p = layout(dst, N)
    zeros = jnp.zeros((N_ACC, H), jnp.float32)
    batch2d = batch.reshape(N, 1)

    h = x
    pools = []
    for i in range(L):
        partials = _sc_scatter(h, src_p, dst_p, zeros)
        t, pool = _tc_dense(partials, h, W1[i], b1[i].reshape(1, H),
                            W2[i], b2[i].reshape(1, H),
                            gamma[i].reshape(1, H), beta[i].reshape(1, H),
                            batch2d)
        pools.append(pool)
        h = t
    return jnp.concatenate(pools, axis=1)
